# Initial kernel scaffold; baseline (speedup 1.0000x reference)
import jax
import jax.numpy as jnp
from jax.experimental import pallas as pl


def kernel(x_s, x_v, edge_attr_s, edge_attr_v, Ws, bs, Wv, Wg, bg, eps, edge_index):
    raise NotImplementedError("write your pallas kernel here")


# speedup vs baseline: 27.6520x; 27.6520x over previous
import functools

import jax
import jax.numpy as jnp
from jax import lax
from jax.experimental import pallas as pl
from jax.experimental.pallas import tpu as pltpu
from jax.experimental.pallas import tpu_sc as plsc

N = 50000
E = 800000
NPAD = 50176
EP = 819200
GT = 400
DUMP = 50000
EW = E // 32
GW = EW // 128
ROWS_T = NPAD // 16

_f32 = jnp.float32


def _node_tables_body(xs_ref, xv_ref, wsi_ref, wsj_ref, wni_ref, wnj_ref,
                      mvi_ref, mvj_ref, gn_ref, dt_ref, st_ref):
    xs = xs_ref[...]
    xv = xv_ref[...]
    nv = jnp.sqrt(jnp.dot(xv * xv, gn_ref[...],
                          preferred_element_type=_f32) + 1e-8)
    dts = (jnp.dot(xs, wsi_ref[...], preferred_element_type=_f32)
           + jnp.dot(nv, wni_ref[...], preferred_element_type=_f32))
    sts = (jnp.dot(xs, wsj_ref[...], preferred_element_type=_f32)
           + jnp.dot(nv, wnj_ref[...], preferred_element_type=_f32))
    dtv = jnp.dot(xv, mvi_ref[...], preferred_element_type=_f32)
    stv = jnp.dot(xv, mvj_ref[...], preferred_element_type=_f32)
    dt_ref[...] = jnp.concatenate([dts, dtv], axis=1)
    st_ref[...] = jnp.concatenate([sts, stv], axis=1)


def _node_tables(xsp, xvfp, wsi, wsj, wni, wnj, mvi, mvj, gn):
    bn = 3136
    grid = NPAD // bn
    full = lambda a: pl.BlockSpec(a.shape, lambda i: (0,) * a.ndim)
    return pl.pallas_call(
        _node_tables_body,
        grid=(grid,),
        in_specs=[
            pl.BlockSpec((bn, 16), lambda i: (i, 0)),
            pl.BlockSpec((bn, 24), lambda i: (i, 0)),
            full(wsi), full(wsj), full(wni), full(wnj),
            full(mvi), full(mvj), full(gn),
        ],
        out_specs=[
            pl.BlockSpec((bn, 40), lambda i: (i, 0)),
            pl.BlockSpec((bn, 40), lambda i: (i, 0)),
        ],
        out_shape=[
            jax.ShapeDtypeStruct((NPAD, 40), _f32),
            jax.ShapeDtypeStruct((NPAD, 40), _f32),
        ],
    )(xsp, xvfp, wsi, wsj, wni, wnj, mvi, mvj, gn)


def _gather_body(dt_h, st_h, dst_h, src_h, ud_h, us_h,
                 ibd, ibs, ga, gb, gta, gtb, sem_g, sem_w):
    c = lax.axis_index("c")
    s = lax.axis_index("s")
    w = s * 2 + c
    ebase = w * EW
    pltpu.sync_copy(dst_h.at[pl.ds(ebase, EW)], ibd.at[pl.ds(0, EW)])
    pltpu.sync_copy(src_h.at[pl.ds(ebase, EW)], ibs.at[pl.ds(0, EW)])

    tail0 = GW * 128
    pltpu.async_copy(dt_h.at[ibd.at[pl.ds(tail0, 40)]], gta, sem_g)
    pltpu.async_copy(st_h.at[ibs.at[pl.ds(tail0, 40)]], gtb, sem_g)
    pltpu.make_async_copy(dt_h.at[ibd.at[pl.ds(tail0, 40)]], gta, sem_g).wait()
    pltpu.make_async_copy(st_h.at[ibs.at[pl.ds(tail0, 40)]], gtb, sem_g).wait()
    pltpu.sync_copy(gta, ud_h.at[pl.ds(ebase + tail0, 40)])
    pltpu.sync_copy(gtb, us_h.at[pl.ds(ebase + tail0, 40)])

    pltpu.async_copy(dt_h.at[ibd.at[pl.ds(0, 128)]], ga.at[0], sem_g)
    pltpu.async_copy(st_h.at[ibs.at[pl.ds(0, 128)]], gb.at[0], sem_g)

    def body(j, carry):
        b = j % 2
        e0 = ebase + j * 128

        @pl.when(j > 0)
        def _():
            e0p = e0 - 128
            pltpu.make_async_copy(ga.at[1 - b], ud_h.at[pl.ds(e0p, 128)],
                                  sem_w).wait()
            pltpu.make_async_copy(gb.at[1 - b], us_h.at[pl.ds(e0p, 128)],
                                  sem_w).wait()

        idx_d = ibd.at[pl.ds(j * 128, 128)]
        idx_s = ibs.at[pl.ds(j * 128, 128)]
        pltpu.make_async_copy(dt_h.at[idx_d], ga.at[b], sem_g).wait()
        pltpu.make_async_copy(st_h.at[idx_s], gb.at[b], sem_g).wait()

        @pl.when(j + 1 < GW)
        def _():
            nxt_d = ibd.at[pl.ds((j + 1) * 128, 128)]
            nxt_s = ibs.at[pl.ds((j + 1) * 128, 128)]
            pltpu.async_copy(dt_h.at[nxt_d], ga.at[1 - b], sem_g)
            pltpu.async_copy(st_h.at[nxt_s], gb.at[1 - b], sem_g)

        pltpu.async_copy(ga.at[b], ud_h.at[pl.ds(e0, 128)], sem_w)
        pltpu.async_copy(gb.at[b], us_h.at[pl.ds(e0, 128)], sem_w)
        return carry

    lax.fori_loop(0, GW, body, 0)
    bl = (GW - 1) % 2
    e0l = ebase + (GW - 1) * 128
    pltpu.make_async_copy(ga.at[bl], ud_h.at[pl.ds(e0l, 128)], sem_w).wait()
    pltpu.make_async_copy(gb.at[bl], us_h.at[pl.ds(e0l, 128)], sem_w).wait()


def _edge_gather(dt, st, dst, src):
    mesh = plsc.VectorSubcoreMesh(core_axis_name="c", subcore_axis_name="s",
                                  num_cores=2, num_subcores=16)
    k = functools.partial(
        pl.kernel,
        out_type=(jax.ShapeDtypeStruct((E, 40), _f32),
                  jax.ShapeDtypeStruct((E, 40), _f32)),
        mesh=mesh,
        compiler_params=pltpu.CompilerParams(use_tc_tiling_on_sc=False),
        scratch_types=[
            pltpu.VMEM((EW,), jnp.int32),
            pltpu.VMEM((EW,), jnp.int32),
            pltpu.VMEM((2, 128, 40), _f32),
            pltpu.VMEM((2, 128, 40), _f32),
            pltpu.VMEM((40, 40), _f32),
            pltpu.VMEM((40, 40), _f32),
            pltpu.SemaphoreType.DMA,
            pltpu.SemaphoreType.DMA,
        ],
    )(_gather_body)
    return k(dt, st, dst, src)


def _edge_msg_body(ud_ref, us_ref, eas_ref, eav_ref, wse_ref, wne_ref,
                   mve_ref, ge_ref, wg_ref, hg_ref, bs_ref, bg_ref, msg_ref):
    u = ud_ref[...] + us_ref[...]
    eas = eas_ref[...]
    eav = eav_ref[...]
    env = jnp.sqrt(jnp.dot(eav * eav, ge_ref[...],
                           preferred_element_type=_f32) + 1e-8)
    cs = (jnp.dot(eas, wse_ref[...], preferred_element_type=_f32)
          + jnp.dot(env, wne_ref[...], preferred_element_type=_f32)
          + bs_ref[...])
    rv = jnp.dot(eav, mve_ref[...], preferred_element_type=_f32)
    s_msg = jnp.maximum(u[:, :16] + cs, 0.0)
    gate = jax.nn.sigmoid(
        jnp.dot(s_msg, wg_ref[...], preferred_element_type=_f32)
        + bg_ref[...])
    g3 = jnp.dot(gate, hg_ref[...], preferred_element_type=_f32)
    v_msg = (u[:, 16:40] + rv) * g3
    z4 = jnp.zeros((s_msg.shape[0], 4), _f32)
    m = jnp.concatenate([s_msg, v_msg], axis=1)
    msg_ref[0] = jnp.concatenate([m[:, :20], z4], axis=1)
    msg_ref[1] = jnp.concatenate([m[:, 20:40], z4], axis=1)


def _edge_msg(ud, us, eas, eav, wse, wne, mve, ge, wg, hg, bs2, bg2):
    be = 8000
    grid = E // be
    full = lambda a: pl.BlockSpec(a.shape, lambda i: (0,) * a.ndim)
    return pl.pallas_call(
        _edge_msg_body,
        grid=(grid,),
        in_specs=[
            pl.BlockSpec((be, 40), lambda i: (i, 0)),
            pl.BlockSpec((be, 40), lambda i: (i, 0)),
            pl.BlockSpec((be, 16), lambda i: (i, 0)),
            pl.BlockSpec((be, 12), lambda i: (i, 0)),
            full(wse), full(wne), full(mve), full(ge), full(wg), full(hg),
            full(bs2), full(bg2),
        ],
        out_specs=pl.BlockSpec((2, be, 24), lambda i: (0, i, 0)),
        out_shape=jax.ShapeDtypeStruct((2, EP, 24), _f32),
    )(ud, us, eas, eav, wse, wne, mve, ge, wg, hg, bs2, bg2)


def _scatter_body(msg_h, d2d_h, zer_h, out_h, ibt, vb, acc, sem_v):
    c = lax.axis_index("c")
    t = lax.axis_index("s")
    g0 = t * GT
    mbase = c * EP + g0 * 128

    pltpu.sync_copy(zer_h, acc.at[pl.ds(t * ROWS_T, ROWS_T)])
    pltpu.sync_copy(d2d_h.at[pl.ds(g0 * 128, GT * 128)], ibt)
    plsc.subcore_barrier()

    def body(j, carry):
        pltpu.sync_copy(msg_h.at[pl.ds(mbase + j * 128, 128)], vb)
        pltpu.sync_copy(vb, acc.at[ibt.at[pl.ds(j * 128, 128)]], add=True)
        return carry

    lax.fori_loop(0, GT, body, 0)

    plsc.subcore_barrier()
    pltpu.sync_copy(acc.at[pl.ds(t * ROWS_T, ROWS_T)],
                    out_h.at[c, pl.ds(t * ROWS_T, ROWS_T)])


def _edge_scatter(msg, d2d, zer):
    mesh = plsc.VectorSubcoreMesh(core_axis_name="c", subcore_axis_name="s",
                                  num_cores=2, num_subcores=16)
    k = functools.partial(
        pl.kernel,
        out_type=jax.ShapeDtypeStruct((2, NPAD, 24), _f32),
        mesh=mesh,
        compiler_params=pltpu.CompilerParams(use_tc_tiling_on_sc=False),
        scratch_types=[
            pltpu.VMEM((GT * 128,), jnp.int32),
            pltpu.VMEM((128, 24), _f32),
            pltpu.VMEM_SHARED((NPAD, 24), _f32),
            pltpu.SemaphoreType.DMA,
        ],
    )(_scatter_body)
    return k(msg, d2d, zer)


def _residual_body(o3_ref, xs_ref, xv_ref, eps_ref, out_ref):
    scale = 1.0 + eps_ref[0, 0]
    acc = jnp.concatenate([o3_ref[0][:, :20], o3_ref[1][:, :20]], axis=1)
    xc = jnp.concatenate([xs_ref[...], xv_ref[...]], axis=1)
    out_ref[...] = acc + scale * xc


def _residual(o3, xsp, xvfp, eps11):
    bn = 3136
    grid = NPAD // bn
    return pl.pallas_call(
        _residual_body,
        grid=(grid,),
        in_specs=[
            pl.BlockSpec((2, bn, 24), lambda i: (0, i, 0)),
            pl.BlockSpec((bn, 16), lambda i: (i, 0)),
            pl.BlockSpec((bn, 24), lambda i: (i, 0)),
            pl.BlockSpec((1, 1), lambda i: (0, 0)),
        ],
        out_specs=pl.BlockSpec((bn, 40), lambda i: (i, 0)),
        out_shape=jax.ShapeDtypeStruct((NPAD, 40), _f32),
    )(o3, xsp, xvfp, eps11)


def kernel(x_s, x_v, edge_attr_s, edge_attr_v, Ws, bs, Wv, Wg, bg, eps,
           edge_index):
    f32 = _f32
    xvf = x_v.reshape(N, 24)
    xsp = jnp.zeros((NPAD, 16), f32).at[:N].set(x_s)
    xvfp = jnp.zeros((NPAD, 24), f32).at[:N].set(xvf)
    eavf = edge_attr_v.reshape(E, 12)

    src = edge_index[0]
    dst = edge_index[1]
    d2d = jnp.full((EP,), DUMP, jnp.int32).at[:E].set(dst)

    eye3 = jnp.eye(3, dtype=f32)
    wsi, wsj, wse = Ws[0:16], Ws[16:32], Ws[32:48]
    wni, wnj, wne = Ws[48:56], Ws[56:64], Ws[64:68]
    mvi = jnp.einsum('ch,de->cdhe', Wv[0:8], eye3).reshape(24, 24)
    mvj = jnp.einsum('ch,de->cdhe', Wv[8:16], eye3).reshape(24, 24)
    mve = jnp.einsum('ch,de->cdhe', Wv[16:20], eye3).reshape(12, 24)
    gn = jnp.repeat(jnp.eye(8, dtype=f32), 3, axis=0)
    ge = jnp.repeat(jnp.eye(4, dtype=f32), 3, axis=0)
    hg = jnp.repeat(jnp.eye(8, dtype=f32), 3, axis=1)
    bs2 = bs.reshape(1, 16)
    bg2 = bg.reshape(1, 8)
    eps11 = eps.reshape(1, 1)
    zer = jnp.zeros((ROWS_T, 24), f32)

    dt, st = _node_tables(xsp, xvfp, wsi, wsj, wni, wnj, mvi, mvj, gn)
    ud, us = _edge_gather(dt, st, dst, src)
    msg = _edge_msg(ud, us, edge_attr_s, eavf, wse, wne, mve, ge, wg=Wg,
                    hg=hg, bs2=bs2, bg2=bg2)
    o3 = _edge_scatter(msg.reshape(2 * EP, 24), d2d, zer)
    outp = _residual(o3, xsp, xvfp, eps11)
    return outp[:N]
